# Initial kernel scaffold; baseline (speedup 1.0000x reference)
import jax
import jax.numpy as jnp
from jax.experimental import pallas as pl


def kernel(top_latent, bottom_latent, top_codebook, bottom_codebook):
    raise NotImplementedError("write your pallas kernel here")


# speedup vs baseline: 2.4257x; 2.4257x over previous
import numpy as np
import jax
import jax.numpy as jnp
from jax import lax
from jax.experimental import pallas as pl
from jax.experimental.pallas import tpu as pltpu

_VOCAB = 1024
_D = 256
_LEVELS = 2
_KL_WEIGHT = 0.001
_ROWS = 8 * 1024
_BN = 512


def _hqvae_block(z_ref, c_ref, out_ref, kl_ref, com_ref):
    z = z_ref[0]
    cb = c_ref[0]
    c2 = jnp.sum(cb * cb, axis=1)
    log_k = np.float32(np.log(float(_VOCAB)))

    r = z
    quant = jnp.zeros_like(z)
    kl_tot = jnp.float32(0.0)
    for _ in range(_LEVELS):
        logits = 2.0 * lax.dot_general(
            r, cb, (((1,), (1,)), ((), ())),
            preferred_element_type=jnp.float32) - c2[None, :]
        m = jnp.max(logits, axis=1, keepdims=True)
        e = jnp.exp(logits - m)
        zsum = jnp.sum(e, axis=1, keepdims=True)
        p = e / zsum
        q = lax.dot_general(
            p, cb, (((1,), (0,)), ((), ())),
            preferred_element_type=jnp.float32)
        plogp = jnp.sum(p * (logits - m), axis=1) - jnp.log(zsum[:, 0])
        kl_tot = kl_tot + jnp.sum(plogp + log_k)
        quant = quant + q
        r = r - q

    out_ref[...] = quant
    kl_ref[...] = kl_tot.reshape(1, 1, 1, 1)
    com_ref[...] = jnp.sum((z - quant) ** 2).reshape(1, 1, 1, 1)


def kernel(top_latent, bottom_latent, top_codebook, bottom_codebook):
    nb = _ROWS // _BN
    zs = jnp.stack([top_latent.reshape(_ROWS, _D),
                    bottom_latent.reshape(_ROWS, _D)])
    cbs = jnp.stack([top_codebook, bottom_codebook])

    quant, kl_parts, com_parts = pl.pallas_call(
        _hqvae_block,
        grid=(2, nb),
        in_specs=[
            pl.BlockSpec((1, _BN, _D), lambda q, i: (q, i, 0)),
            pl.BlockSpec((1, _VOCAB, _D), lambda q, i: (q, 0, 0)),
        ],
        out_specs=(
            pl.BlockSpec((_BN, _D), lambda q, i: (i, q)),
            pl.BlockSpec((1, 1, 1, 1), lambda q, i: (q, i, 0, 0)),
            pl.BlockSpec((1, 1, 1, 1), lambda q, i: (q, i, 0, 0)),
        ),
        out_shape=(
            jax.ShapeDtypeStruct((_ROWS, 2 * _D), jnp.float32),
            jax.ShapeDtypeStruct((2, nb, 1, 1), jnp.float32),
            jax.ShapeDtypeStruct((2, nb, 1, 1), jnp.float32),
        ),
    )(zs, cbs)

    loss = (jnp.sum(com_parts) / np.float32(_ROWS * _D)
            + np.float32(_KL_WEIGHT) * jnp.sum(kl_parts) / np.float32(_ROWS))
    return (loss, quant.reshape(8, 1024, 2 * _D))
